# baseline (speedup 1.0000x reference)
import functools

import jax
import jax.numpy as jnp
from jax import lax
from jax.experimental import pallas as pl
from jax.experimental.pallas import tpu as pltpu
from jax.experimental.pallas import tpu_sc as plsc

N = 10000
E = 320000
G = 128
H = 64
HX = 80
N_PAD = 10240
B = 1024
NB = N_PAD // B
NSUB = 16
NCORE = 2
NW = NSUB * NCORE
EPW = E // NW
K = 128
NCHUNK = -(-EPW // K)
EPW_PAD = NCHUNK * K
TRASH = N_PAD - 2
ROWS_PT = N_PAD // NSUB

_f32 = jnp.float32


def _dense_common(h, aux_ref, i, h_ref, als_ref, ald_ref, cs_ref, cd_ref):
    a_s = aux_ref[0, :]
    a_d = aux_ref[1, :]
    als = jnp.sum(h * a_s[None, :], axis=1)
    ald = jnp.sum(h * a_d[None, :], axis=1)
    h_ref[...] = jnp.concatenate(
        [h, jnp.ones((B, 1), _f32), jnp.zeros((B, HX - H - 1), _f32)], axis=1)
    als_ref[...] = als
    ald_ref[...] = ald

    @pl.when(i == 0)
    def _():
        cs_ref[...] = jnp.full((8, 128), -1e30, _f32)
        cd_ref[...] = jnp.full((8, 128), -1e30, _f32)

    cs_ref[...] = jnp.maximum(cs_ref[...], jnp.max(als))
    cd_ref[...] = jnp.maximum(cd_ref[...], jnp.max(ald))


def _dense1_body(x_ref, w_ref, aux_ref, h_ref, als_ref, ald_ref, cs_ref, cd_ref):
    i = pl.program_id(0)
    h = jnp.dot(x_ref[...], w_ref[...], preferred_element_type=_f32)
    _dense_common(h, aux_ref, i, h_ref, als_ref, ald_ref, cs_ref, cd_ref)


def _dense23_body(p_ref, auxp_ref, w_ref, aux_ref, h_ref, als_ref, ald_ref,
                  cs_ref, cd_ref):
    i = pl.program_id(0)
    p = p_ref[...]
    num = p[0, :, :H] + p[1, :, :H]
    den = p[0, :, H:H + 1] + p[1, :, H:H + 1]
    bprev = auxp_ref[2, :]
    x = jnp.maximum(num / den + bprev[None, :], 0.0)
    h = jnp.dot(x, w_ref[...], preferred_element_type=_f32)
    _dense_common(h, aux_ref, i, h_ref, als_ref, ald_ref, cs_ref, cd_ref)


_dense_outs = [
    jax.ShapeDtypeStruct((N_PAD, HX), _f32),
    jax.ShapeDtypeStruct((N_PAD,), _f32),
    jax.ShapeDtypeStruct((N_PAD,), _f32),
    jax.ShapeDtypeStruct((8, 128), _f32),
    jax.ShapeDtypeStruct((8, 128), _f32),
]
_dense_out_specs = [
    pl.BlockSpec((B, HX), lambda i: (i, 0)),
    pl.BlockSpec((B,), lambda i: (i,)),
    pl.BlockSpec((B,), lambda i: (i,)),
    pl.BlockSpec((8, 128), lambda i: (0, 0)),
    pl.BlockSpec((8, 128), lambda i: (0, 0)),
]

_dense1_call = pl.pallas_call(
    _dense1_body,
    grid=(NB,),
    in_specs=[
        pl.BlockSpec((B, 128), lambda i: (i, 0)),
        pl.BlockSpec((128, H), lambda i: (0, 0)),
        pl.BlockSpec((8, H), lambda i: (0, 0)),
    ],
    out_specs=_dense_out_specs,
    out_shape=_dense_outs,
)

_dense23_call = pl.pallas_call(
    _dense23_body,
    grid=(NB,),
    in_specs=[
        pl.BlockSpec((NCORE, B, HX), lambda i: (0, i, 0)),
        pl.BlockSpec((8, H), lambda i: (0, 0)),
        pl.BlockSpec((H, H), lambda i: (0, 0)),
        pl.BlockSpec((8, H), lambda i: (0, 0)),
    ],
    out_specs=_dense_out_specs,
    out_shape=_dense_outs,
)


_mesh = plsc.VectorSubcoreMesh(
    core_axis_name="c", subcore_axis_name="s",
    num_cores=NCORE, num_subcores=NSUB)


@functools.partial(
    pl.kernel,
    mesh=_mesh,
    compiler_params=pltpu.CompilerParams(
        needs_layout_passes=False, use_tc_tiling_on_sc=False),
    out_type=jax.ShapeDtypeStruct((NCORE, N_PAD, HX), _f32),
    scratch_types=[
        pltpu.VMEM((N_PAD,), _f32),
        pltpu.VMEM((N_PAD,), _f32),
        pltpu.VMEM((16,), _f32),
        pltpu.VMEM((NCHUNK, K), jnp.int32),
        pltpu.VMEM((NCHUNK, K), jnp.int32),
        pltpu.VMEM((2, K, HX), _f32),
        pltpu.VMEM_SHARED((N_PAD, HX), _f32),
        pltpu.SemaphoreType.DMA,
        pltpu.SemaphoreType.DMA,
        pltpu.SemaphoreType.DMA,
        pltpu.SemaphoreType.DMA,
    ],
)
def _sc_edge(h_hbm, als_hbm, ald_hbm, c_hbm, src_hbm, dst_hbm, out_hbm,
             als_v, ald_v, cv, srcb, dstb, rows, num_sh,
             gsem0, gsem1, ssem0, ssem1):
    cid = lax.axis_index("c")
    sid = lax.axis_index("s")
    wid = cid * NSUB + sid
    r0 = sid * ROWS_PT
    gsem = (gsem0, gsem1)
    ssem = (ssem0, ssem1)

    pltpu.sync_copy(als_hbm, als_v)
    pltpu.sync_copy(ald_hbm, ald_v)
    pltpu.sync_copy(c_hbm, cv)
    cvec = cv[...]

    do_init = (cid == 0) == (sid < NSUB // 2)

    @pl.when(do_init)
    def _():
        def seed(bq, carry):
            rr = r0 + bq * K
            pltpu.sync_copy(h_hbm.at[pl.ds(rr, K)], rows.at[0])
            for v in range(K // 16):
                a = als_v[pl.ds(rr + v * 16, 16)]
                b = ald_v[pl.ds(rr + v * 16, 16)]
                z = a + b
                e = jnp.maximum(z, 0.2 * z)
                w = jnp.exp(e - cvec)
                for t in range(16):
                    s = w[t]
                    jj = v * 16 + t
                    for q in range(HX // 16):
                        rows[0, jj, pl.ds(q * 16, 16)] = (
                            rows[0, jj, pl.ds(q * 16, 16)] * s)
            pltpu.sync_copy(rows.at[0], num_sh.at[pl.ds(rr, K)])
            return carry

        lax.fori_loop(0, ROWS_PT // K, seed, 0)

    @pl.when(jnp.logical_not(do_init))
    def _():
        zero16 = jnp.zeros((16,), _f32)
        for jj in range(K):
            for q in range(HX // 16):
                rows[0, jj, pl.ds(q * 16, 16)] = zero16

        def zseed(bq, carry):
            pltpu.sync_copy(rows.at[0], num_sh.at[pl.ds(r0 + bq * K, K)])
            return carry

        lax.fori_loop(0, ROWS_PT // K, zseed, 0)

    plsc.subcore_barrier()
    cbase = wid * NCHUNK
    pltpu.sync_copy(src_hbm.at[pl.ds(cbase, NCHUNK)], srcb)
    pltpu.sync_copy(dst_hbm.at[pl.ds(cbase, NCHUNK)], dstb)

    def scale(j, p):
        for v in range(K // 16):
            sv = srcb[j, pl.ds(v * 16, 16)]
            dv = dstb[j, pl.ds(v * 16, 16)]
            a = plsc.load_gather(als_v, [sv])
            b = plsc.load_gather(ald_v, [dv])
            z = a + b
            e = jnp.maximum(z, 0.2 * z)
            w = jnp.exp(e - cvec)
            for t in range(16):
                s = w[t]
                jj = v * 16 + t
                for q in range(HX // 16):
                    rows[p, jj, pl.ds(q * 16, 16)] = (
                        rows[p, jj, pl.ds(q * 16, 16)] * s)

    def half(j, p, prefetch):
        np_ = 1 - p
        pltpu.make_async_copy(h_hbm.at[srcb.at[j]], rows.at[p],
                              gsem[p]).wait()
        scale(j, p)

        @pl.when(j > 0)
        def _():
            pltpu.make_async_copy(rows.at[np_], num_sh.at[dstb.at[j - 1]],
                                  ssem[np_]).wait()

        if prefetch:
            pltpu.async_copy(h_hbm.at[srcb.at[j + 1]], rows.at[np_],
                             gsem[np_])
        pltpu.async_copy(rows.at[p], num_sh.at[dstb.at[j]], ssem[p],
                         add=True)

    pltpu.async_copy(h_hbm.at[srcb.at[0]], rows.at[0], gsem[0])

    def pair(i, carry):
        half(2 * i, 0, True)
        half(2 * i + 1, 1, True)
        return carry

    lax.fori_loop(0, (NCHUNK - 1) // 2, pair, 0)
    half(NCHUNK - 1, 0, False)
    pltpu.make_async_copy(rows.at[0], num_sh.at[dstb.at[NCHUNK - 1]],
                          ssem[0]).wait()
    plsc.subcore_barrier()
    pltpu.sync_copy(num_sh.at[pl.ds(r0, ROWS_PT)],
                    out_hbm.at[cid, pl.ds(r0, ROWS_PT)])


def _pool_body(p_ref, aux3_ref, batch_ref, l1w_ref, head_ref, out_ref,
               acc_ref):
    i = pl.program_id(0)
    p = p_ref[...]
    num = p[0, :, :H] + p[1, :, :H]
    den = p[0, :, H:H + 1] + p[1, :, H:H + 1]
    b3 = aux3_ref[2, :]
    x = jnp.maximum(num / den + b3[None, :], 0.0)
    xe = jnp.concatenate([x, jnp.ones((B, 1), _f32)], axis=1)
    bt = batch_ref[...]
    gids = lax.broadcasted_iota(jnp.int32, (B, G), 1)
    oh = (bt[:, None] == gids).astype(_f32)
    part = lax.dot_general(oh, xe, (((0,), (0,)), ((), ())),
                           preferred_element_type=_f32)

    @pl.when(i == 0)
    def _():
        acc_ref[...] = jnp.zeros((G, H + 1), _f32)

    acc_ref[...] += part

    @pl.when(i == NB - 1)
    def _():
        acc = acc_ref[...]
        g = acc[:, :H] / jnp.maximum(acc[:, H:H + 1], 1.0)
        a1 = jnp.maximum(
            jnp.dot(g, l1w_ref[...], preferred_element_type=_f32)
            + head_ref[0:1, :], 0.0)
        zz = jnp.sum(a1 * head_ref[1:2, :], axis=1, keepdims=True)
        zz = zz + head_ref[2, 0]
        out_ref[...] = 1.0 / (1.0 + jnp.exp(-zz))


_pool_call = pl.pallas_call(
    _pool_body,
    grid=(NB,),
    in_specs=[
        pl.BlockSpec((NCORE, B, HX), lambda i: (0, i, 0)),
        pl.BlockSpec((8, H), lambda i: (0, 0)),
        pl.BlockSpec((B,), lambda i: (i,)),
        pl.BlockSpec((H, H), lambda i: (0, 0)),
        pl.BlockSpec((8, H), lambda i: (0, 0)),
    ],
    out_specs=pl.BlockSpec((G, 1), lambda i: (0, 0)),
    out_shape=jax.ShapeDtypeStruct((G, 1), _f32),
    scratch_shapes=[pltpu.VMEM((G, H + 1), _f32)],
)


def _aux(a_s, a_d, b):
    return jnp.stack(
        [a_s, a_d, b] + [jnp.zeros((H,), _f32)] * 5, axis=0)


def kernel(x, edge_index, batch, W1, a_src1, a_dst1, b1, W2, a_src2, a_dst2,
           b2, W3, a_src3, a_dst3, b3, lin1_W, lin1_b, lin2_W, lin2_b):
    pad = EPW_PAD - EPW
    src = jnp.concatenate(
        [edge_index[0].reshape(NW, EPW),
         jnp.zeros((NW, pad), jnp.int32)], axis=1).reshape(-1, K)
    dst = jnp.concatenate(
        [edge_index[1].reshape(NW, EPW),
         jnp.full((NW, pad), TRASH, jnp.int32)], axis=1).reshape(-1, K)
    xp = jnp.zeros((N_PAD, 128), _f32).at[:N].set(x)
    batchp = jnp.full((N_PAD,), G, jnp.int32).at[:N].set(batch)
    aux1 = _aux(a_src1, a_dst1, b1)
    aux2 = _aux(a_src2, a_dst2, b2)
    aux3 = _aux(a_src3, a_dst3, b3)
    head = jnp.stack(
        [lin1_b, lin2_W[:, 0],
         jnp.zeros((H,), _f32).at[0].set(lin2_b[0])]
        + [jnp.zeros((H,), _f32)] * 5, axis=0)

    h_ext, als, ald, cs, cd = _dense1_call(xp, W1, aux1)
    for (Wl, auxl, auxp) in ((W2, aux2, aux1), (W3, aux3, aux2)):
        c16 = cs[0, :16] + cd[0, :16]
        p = _sc_edge(h_ext, als, ald, c16, src, dst)
        h_ext, als, ald, cs, cd = _dense23_call(p, auxp, Wl, auxl)

    c16 = cs[0, :16] + cd[0, :16]
    p = _sc_edge(h_ext, als, ald, c16, src, dst)

    return _pool_call(p, aux3, batchp, lin1_W, head)


# speedup vs baseline: 1.6144x; 1.6144x over previous
import functools

import jax
import jax.numpy as jnp
from jax import lax
from jax.experimental import pallas as pl
from jax.experimental.pallas import tpu as pltpu
from jax.experimental.pallas import tpu_sc as plsc

N = 10000
E = 320000
G = 128
H = 64
HX = 80
N_PAD = 10240
B = 1024
NB = N_PAD // B
NSUB = 16
NCORE = 2
NW = NSUB * NCORE
EPW = E // NW
K = 80
NCHUNK = -(-EPW // K)
EPW_PAD = NCHUNK * K
TRASH = N_PAD - 2
ROWS_PT = N_PAD // NSUB

_f32 = jnp.float32


def _dense_common(h, aux_ref, i, h_ref, als_ref, ald_ref, cs_ref, cd_ref):
    a_s = aux_ref[0, :]
    a_d = aux_ref[1, :]
    als = jnp.sum(h * a_s[None, :], axis=1)
    ald = jnp.sum(h * a_d[None, :], axis=1)
    h_ref[...] = jnp.concatenate(
        [h, jnp.ones((B, 1), _f32), jnp.zeros((B, HX - H - 1), _f32)], axis=1)
    als_ref[...] = als
    ald_ref[...] = ald

    @pl.when(i == 0)
    def _():
        cs_ref[...] = jnp.full((8, 128), -1e30, _f32)
        cd_ref[...] = jnp.full((8, 128), -1e30, _f32)

    cs_ref[...] = jnp.maximum(cs_ref[...], jnp.max(als))
    cd_ref[...] = jnp.maximum(cd_ref[...], jnp.max(ald))


def _dense1_body(x_ref, w_ref, aux_ref, h_ref, als_ref, ald_ref, cs_ref, cd_ref):
    i = pl.program_id(0)
    h = jnp.dot(x_ref[...], w_ref[...], preferred_element_type=_f32)
    _dense_common(h, aux_ref, i, h_ref, als_ref, ald_ref, cs_ref, cd_ref)


def _dense23_body(p_ref, auxp_ref, w_ref, aux_ref, h_ref, als_ref, ald_ref,
                  cs_ref, cd_ref):
    i = pl.program_id(0)
    p = p_ref[...]
    num = p[0, :, :H] + p[1, :, :H]
    den = p[0, :, H:H + 1] + p[1, :, H:H + 1]
    bprev = auxp_ref[2, :]
    x = jnp.maximum(num / den + bprev[None, :], 0.0)
    h = jnp.dot(x, w_ref[...], preferred_element_type=_f32)
    _dense_common(h, aux_ref, i, h_ref, als_ref, ald_ref, cs_ref, cd_ref)


_dense_outs = [
    jax.ShapeDtypeStruct((N_PAD, HX), _f32),
    jax.ShapeDtypeStruct((N_PAD,), _f32),
    jax.ShapeDtypeStruct((N_PAD,), _f32),
    jax.ShapeDtypeStruct((8, 128), _f32),
    jax.ShapeDtypeStruct((8, 128), _f32),
]
_dense_out_specs = [
    pl.BlockSpec((B, HX), lambda i: (i, 0)),
    pl.BlockSpec((B,), lambda i: (i,)),
    pl.BlockSpec((B,), lambda i: (i,)),
    pl.BlockSpec((8, 128), lambda i: (0, 0)),
    pl.BlockSpec((8, 128), lambda i: (0, 0)),
]

_dense1_call = pl.pallas_call(
    _dense1_body,
    grid=(NB,),
    in_specs=[
        pl.BlockSpec((B, 128), lambda i: (i, 0)),
        pl.BlockSpec((128, H), lambda i: (0, 0)),
        pl.BlockSpec((8, H), lambda i: (0, 0)),
    ],
    out_specs=_dense_out_specs,
    out_shape=_dense_outs,
)

_dense23_call = pl.pallas_call(
    _dense23_body,
    grid=(NB,),
    in_specs=[
        pl.BlockSpec((NCORE, B, HX), lambda i: (0, i, 0)),
        pl.BlockSpec((8, H), lambda i: (0, 0)),
        pl.BlockSpec((H, H), lambda i: (0, 0)),
        pl.BlockSpec((8, H), lambda i: (0, 0)),
    ],
    out_specs=_dense_out_specs,
    out_shape=_dense_outs,
)


_mesh = plsc.VectorSubcoreMesh(
    core_axis_name="c", subcore_axis_name="s",
    num_cores=NCORE, num_subcores=NSUB)


@functools.partial(
    pl.kernel,
    mesh=_mesh,
    compiler_params=pltpu.CompilerParams(
        needs_layout_passes=False, use_tc_tiling_on_sc=False),
    out_type=jax.ShapeDtypeStruct((NCORE, N_PAD, HX), _f32),
    scratch_types=[
        pltpu.VMEM((N_PAD,), _f32),
        pltpu.VMEM((N_PAD,), _f32),
        pltpu.VMEM((16,), _f32),
        pltpu.VMEM((NCHUNK, K), jnp.int32),
        pltpu.VMEM((NCHUNK, K), jnp.int32),
        pltpu.VMEM((4, K, HX), _f32),
        pltpu.VMEM_SHARED((N_PAD, HX), _f32),
        pltpu.SemaphoreType.DMA,
        pltpu.SemaphoreType.DMA,
        pltpu.SemaphoreType.DMA,
        pltpu.SemaphoreType.DMA,
        pltpu.SemaphoreType.DMA,
        pltpu.SemaphoreType.DMA,
        pltpu.SemaphoreType.DMA,
        pltpu.SemaphoreType.DMA,
    ],
)
def _sc_edge(h_hbm, als_hbm, ald_hbm, c_hbm, src_hbm, dst_hbm, out_hbm,
             als_v, ald_v, cv, srcb, dstb, rows, num_sh,
             gsem0, gsem1, gsem2, gsem3, ssem0, ssem1, ssem2, ssem3):
    cid = lax.axis_index("c")
    sid = lax.axis_index("s")
    wid = cid * NSUB + sid
    r0 = sid * ROWS_PT
    gsem = (gsem0, gsem1, gsem2, gsem3)
    ssem = (ssem0, ssem1, ssem2, ssem3)

    pltpu.sync_copy(als_hbm, als_v)
    pltpu.sync_copy(ald_hbm, ald_v)
    pltpu.sync_copy(c_hbm, cv)
    cvec = cv[...]

    do_init = (cid == 0) == (sid < NSUB // 2)

    @pl.when(do_init)
    def _():
        def seed(bq, carry):
            rr = r0 + bq * K
            pltpu.sync_copy(h_hbm.at[pl.ds(rr, K)], rows.at[0])
            for v in range(K // 16):
                a = als_v[pl.ds(rr + v * 16, 16)]
                b = ald_v[pl.ds(rr + v * 16, 16)]
                z = a + b
                e = jnp.maximum(z, 0.2 * z)
                w = jnp.exp(e - cvec)
                for t in range(16):
                    s = w[t]
                    jj = v * 16 + t
                    for q in range(HX // 16):
                        rows[0, jj, pl.ds(q * 16, 16)] = (
                            rows[0, jj, pl.ds(q * 16, 16)] * s)
            pltpu.sync_copy(rows.at[0], num_sh.at[pl.ds(rr, K)])
            return carry

        lax.fori_loop(0, ROWS_PT // K, seed, 0)

    @pl.when(jnp.logical_not(do_init))
    def _():
        zero16 = jnp.zeros((16,), _f32)
        for jj in range(K):
            for q in range(HX // 16):
                rows[0, jj, pl.ds(q * 16, 16)] = zero16

        def zseed(bq, carry):
            pltpu.sync_copy(rows.at[0], num_sh.at[pl.ds(r0 + bq * K, K)])
            return carry

        lax.fori_loop(0, ROWS_PT // K, zseed, 0)

    plsc.subcore_barrier()
    cbase = wid * NCHUNK
    pltpu.sync_copy(src_hbm.at[pl.ds(cbase, NCHUNK)], srcb)
    pltpu.sync_copy(dst_hbm.at[pl.ds(cbase, NCHUNK)], dstb)

    def scale(j, p):
        for v in range(K // 16):
            sv = srcb[j, pl.ds(v * 16, 16)]
            dv = dstb[j, pl.ds(v * 16, 16)]
            a = plsc.load_gather(als_v, [sv])
            b = plsc.load_gather(ald_v, [dv])
            z = a + b
            e = jnp.maximum(z, 0.2 * z)
            w = jnp.exp(e - cvec)
            for t in range(16):
                s = w[t]
                jj = v * 16 + t
                for q in range(HX // 16):
                    rows[p, jj, pl.ds(q * 16, 16)] = (
                        rows[p, jj, pl.ds(q * 16, 16)] * s)

    def step(j, b, prefetch):
        b1 = (b + 1) % 4
        if prefetch:
            @pl.when(j >= 3)
            def _():
                pltpu.make_async_copy(rows.at[b1],
                                      num_sh.at[dstb.at[j - 3]],
                                      ssem[b1]).wait()

            pltpu.async_copy(h_hbm.at[srcb.at[j + 1]], rows.at[b1],
                             gsem[b1])
        pltpu.make_async_copy(h_hbm.at[srcb.at[j]], rows.at[b],
                              gsem[b]).wait()
        scale(j, b)
        pltpu.async_copy(rows.at[b], num_sh.at[dstb.at[j]], ssem[b],
                         add=True)

    pltpu.async_copy(h_hbm.at[srcb.at[0]], rows.at[0], gsem[0])

    def quad(i, carry):
        for b in range(4):
            step(4 * i + b, b, True)
        return carry

    lax.fori_loop(0, NCHUNK // 4, quad, 0)
    for jt in range(NCHUNK - (NCHUNK // 4) * 4):
        j = (NCHUNK // 4) * 4 + jt
        step(j, j % 4, j + 1 < NCHUNK)
    for j in range(NCHUNK - 4, NCHUNK):
        pltpu.make_async_copy(rows.at[j % 4], num_sh.at[dstb.at[j]],
                              ssem[j % 4]).wait()
    plsc.subcore_barrier()
    pltpu.sync_copy(num_sh.at[pl.ds(r0, ROWS_PT)],
                    out_hbm.at[cid, pl.ds(r0, ROWS_PT)])


def _pool_body(p_ref, aux3_ref, batch_ref, l1w_ref, head_ref, out_ref,
               acc_ref):
    i = pl.program_id(0)
    p = p_ref[...]
    num = p[0, :, :H] + p[1, :, :H]
    den = p[0, :, H:H + 1] + p[1, :, H:H + 1]
    b3 = aux3_ref[2, :]
    x = jnp.maximum(num / den + b3[None, :], 0.0)
    xe = jnp.concatenate([x, jnp.ones((B, 1), _f32)], axis=1)
    bt = batch_ref[...]
    gids = lax.broadcasted_iota(jnp.int32, (B, G), 1)
    oh = (bt[:, None] == gids).astype(_f32)
    part = lax.dot_general(oh, xe, (((0,), (0,)), ((), ())),
                           preferred_element_type=_f32)

    @pl.when(i == 0)
    def _():
        acc_ref[...] = jnp.zeros((G, H + 1), _f32)

    acc_ref[...] += part

    @pl.when(i == NB - 1)
    def _():
        acc = acc_ref[...]
        g = acc[:, :H] / jnp.maximum(acc[:, H:H + 1], 1.0)
        a1 = jnp.maximum(
            jnp.dot(g, l1w_ref[...], preferred_element_type=_f32)
            + head_ref[0:1, :], 0.0)
        zz = jnp.sum(a1 * head_ref[1:2, :], axis=1, keepdims=True)
        zz = zz + head_ref[2, 0]
        out_ref[...] = 1.0 / (1.0 + jnp.exp(-zz))


_pool_call = pl.pallas_call(
    _pool_body,
    grid=(NB,),
    in_specs=[
        pl.BlockSpec((NCORE, B, HX), lambda i: (0, i, 0)),
        pl.BlockSpec((8, H), lambda i: (0, 0)),
        pl.BlockSpec((B,), lambda i: (i,)),
        pl.BlockSpec((H, H), lambda i: (0, 0)),
        pl.BlockSpec((8, H), lambda i: (0, 0)),
    ],
    out_specs=pl.BlockSpec((G, 1), lambda i: (0, 0)),
    out_shape=jax.ShapeDtypeStruct((G, 1), _f32),
    scratch_shapes=[pltpu.VMEM((G, H + 1), _f32)],
)


def _aux(a_s, a_d, b):
    return jnp.stack(
        [a_s, a_d, b] + [jnp.zeros((H,), _f32)] * 5, axis=0)


def kernel(x, edge_index, batch, W1, a_src1, a_dst1, b1, W2, a_src2, a_dst2,
           b2, W3, a_src3, a_dst3, b3, lin1_W, lin1_b, lin2_W, lin2_b):
    pad = EPW_PAD - EPW
    src = jnp.concatenate(
        [edge_index[0].reshape(NW, EPW),
         jnp.zeros((NW, pad), jnp.int32)], axis=1).reshape(-1, K)
    dst = jnp.concatenate(
        [edge_index[1].reshape(NW, EPW),
         jnp.full((NW, pad), TRASH, jnp.int32)], axis=1).reshape(-1, K)
    xp = jnp.zeros((N_PAD, 128), _f32).at[:N].set(x)
    batchp = jnp.full((N_PAD,), G, jnp.int32).at[:N].set(batch)
    aux1 = _aux(a_src1, a_dst1, b1)
    aux2 = _aux(a_src2, a_dst2, b2)
    aux3 = _aux(a_src3, a_dst3, b3)
    head = jnp.stack(
        [lin1_b, lin2_W[:, 0],
         jnp.zeros((H,), _f32).at[0].set(lin2_b[0])]
        + [jnp.zeros((H,), _f32)] * 5, axis=0)

    h_ext, als, ald, cs, cd = _dense1_call(xp, W1, aux1)
    for (Wl, auxl, auxp) in ((W2, aux2, aux1), (W3, aux3, aux2)):
        c16 = cs[0, :16] + cd[0, :16]
        p = _sc_edge(h_ext, als, ald, c16, src, dst)
        h_ext, als, ald, cs, cd = _dense23_call(p, auxp, Wl, auxl)

    c16 = cs[0, :16] + cd[0, :16]
    p = _sc_edge(h_ext, als, ald, c16, src, dst)

    return _pool_call(p, aux3, batchp, lin1_W, head)
